# baseline (speedup 1.0000x reference)
import functools
import jax
import jax.numpy as jnp
from jax import lax
from jax.experimental import pallas as pl
from jax.experimental.pallas import tpu as pltpu
from jax.experimental.pallas import tpu_sc as plsc

N = 10000
E = 320000
D = 128
H = 128
C = 40
F2 = 48

NC, NS = 2, 16
NW = NC * NS
CH = 40
K = E // (NW * CH)
SEC = 5
KS = K // SEC
NB = 5
NPAD = 10240
RPT = NPAD // NS

_SC_MESH = dict(core_axis_name="c", subcore_axis_name="s")


@functools.partial(
    pl.kernel,
    out_type=jax.ShapeDtypeStruct((NC, NPAD), jnp.float32),
    mesh=plsc.VectorSubcoreMesh(**_SC_MESH),
    scratch_types=[
        pltpu.VMEM((K, CH), jnp.int32),
        pltpu.VMEM((RPT,), jnp.float32),
        pltpu.VMEM_SHARED((NPAD,), jnp.float32),
        pltpu.SemaphoreType.DMA,
    ],
)
def _sc_degree(colr, out, idx_v, ones_v, acc, sem):
    cid = lax.axis_index("c")
    sid = lax.axis_index("s")
    wid = sid * NC + cid
    r0 = sid * RPT

    def fill(i, carry):
        ones_v[pl.ds(i * 16, 16)] = jnp.zeros((16,), jnp.float32)
        return carry

    lax.fori_loop(0, RPT // 16, fill, 0)
    pltpu.sync_copy(ones_v, acc.at[pl.ds(r0, RPT)])
    for i in range((CH + 15) // 16):
        ones_v[pl.ds(i * 16, 16)] = jnp.full((16,), 1.0, jnp.float32)
    pltpu.sync_copy(colr.at[wid], idx_v)
    plsc.subcore_barrier()

    def fire(j, carry):
        pltpu.async_copy(ones_v.at[pl.ds(0, CH)], acc.at[idx_v.at[j]], sem,
                         add=True)
        return carry

    def drain(j, carry):
        pltpu.make_async_copy(ones_v.at[pl.ds(0, CH)], acc.at[idx_v.at[0]],
                              sem).wait()
        return carry

    lax.fori_loop(0, K, fire, 0)
    lax.fori_loop(0, K, drain, 0)
    plsc.subcore_barrier()
    pltpu.sync_copy(acc.at[pl.ds(r0, RPT)], out.at[cid, pl.ds(r0, RPT)])


def _make_agg(F, tc_tiling=True):
    @functools.partial(
        pl.kernel,
        out_type=jax.ShapeDtypeStruct((NC, NPAD, F), jnp.float32),
        mesh=plsc.VectorSubcoreMesh(**_SC_MESH),
        compiler_params=pltpu.CompilerParams(use_tc_tiling_on_sc=tc_tiling),
        scratch_types=[
            pltpu.VMEM((KS, CH), jnp.int32),
            pltpu.VMEM((KS, CH), jnp.int32),
        ] + [pltpu.VMEM((CH, F), jnp.float32) for _ in range(NB)]
          + [pltpu.VMEM_SHARED((NPAD, F), jnp.float32)]
          + [pltpu.SemaphoreType.DMA for _ in range(2 * NB)],
    )
    def agg(h, rowr, colr, out, row_v, col_v, *rest):
        bufs = rest[:NB]
        acc = rest[NB]
        gs = rest[NB + 1:NB + 1 + NB]
        ss = rest[NB + 1 + NB:]
        cid = lax.axis_index("c")
        sid = lax.axis_index("s")
        wid = sid * NC + cid
        r0 = sid * RPT

        def fill(i, carry):
            bufs[0][i // (F // 16), pl.ds((i % (F // 16)) * 16, 16)] = \
                jnp.zeros((16,), jnp.float32)
            return carry

        lax.fori_loop(0, CH * F // 16, fill, 0)
        for jj in range(RPT // CH):
            pltpu.sync_copy(bufs[0], acc.at[pl.ds(r0 + jj * CH, CH)])
        plsc.subcore_barrier()

        for s in range(SEC):
            pltpu.sync_copy(rowr.at[wid, s], row_v)
            pltpu.sync_copy(colr.at[wid, s], col_v)
            for i in range(NB):
                pltpu.async_copy(h.at[row_v.at[i]], bufs[i], gs[i])

            def body(g, carry):
                base = NB * g
                for i in range(NB):
                    j = base + i
                    pltpu.make_async_copy(h.at[row_v.at[j]], bufs[i],
                                          gs[i]).wait()
                    pltpu.async_copy(bufs[i], acc.at[col_v.at[j]], ss[i],
                                     add=True)
                for i in range(NB):
                    j = base + NB + i
                    pltpu.make_async_copy(bufs[i], acc.at[col_v.at[j]],
                                          ss[i]).wait()
                    pltpu.async_copy(h.at[row_v.at[j]], bufs[i], gs[i])
                return carry

            lax.fori_loop(0, KS // NB - 1, body, 0)
            for i in range(NB):
                j = KS - NB + i
                pltpu.make_async_copy(h.at[row_v.at[j]], bufs[i],
                                      gs[i]).wait()
                pltpu.async_copy(bufs[i], acc.at[col_v.at[j]], ss[i],
                                 add=True)
            for i in range(NB):
                pltpu.make_async_copy(bufs[i], acc.at[col_v.at[KS - 1]],
                                      ss[i]).wait()
        plsc.subcore_barrier()
        pltpu.sync_copy(acc.at[pl.ds(r0, RPT)],
                        out.at[cid, pl.ds(r0, RPT)])

    return agg


_sc_agg_h = _make_agg(H)
_sc_agg_c = _make_agg(F2, tc_tiling=False)


_BR = 1000


def _tc_mm_raw(x, W0):
    def body(x_ref, w_ref, h_ref):
        h_ref[...] = jnp.dot(x_ref[...], w_ref[...],
                             preferred_element_type=jnp.float32)

    return pl.pallas_call(
        body,
        grid=(N // _BR,),
        in_specs=[
            pl.BlockSpec((_BR, D), lambda i: (i, 0)),
            pl.BlockSpec((D, H), lambda i: (0, 0)),
        ],
        out_specs=pl.BlockSpec((_BR, H), lambda i: (i, 0)),
        out_shape=jax.ShapeDtypeStruct((N, H), jnp.float32),
    )(x, W0)


def _tc_scale(h0, ca, cb):
    def body(h0_ref, ca_ref, cb_ref, h_ref, d_ref):
        d = lax.rsqrt(ca_ref[...] + cb_ref[...] + 1.0)
        h_ref[...] = h0_ref[...] * d
        d_ref[...] = d

    return pl.pallas_call(
        body,
        grid=(N // _BR,),
        in_specs=[
            pl.BlockSpec((_BR, H), lambda i: (i, 0)),
            pl.BlockSpec((_BR, 1), lambda i: (i, 0)),
            pl.BlockSpec((_BR, 1), lambda i: (i, 0)),
        ],
        out_specs=[
            pl.BlockSpec((_BR, H), lambda i: (i, 0)),
            pl.BlockSpec((_BR, 1), lambda i: (i, 0)),
        ],
        out_shape=[
            jax.ShapeDtypeStruct((N, H), jnp.float32),
            jax.ShapeDtypeStruct((N, 1), jnp.float32),
        ],
    )(h0, ca, cb)


def _tc_mm2(aa, ab, h0p, dinv, b0, W1p):
    def body(aa_ref, ab_ref, h_ref, d_ref, b_ref, w_ref, o_ref):
        d = d_ref[...]
        z = jnp.maximum((aa_ref[...] + ab_ref[...] + h_ref[...]) * d
                        + b_ref[...], 0.0)
        o_ref[...] = jnp.dot(z, w_ref[...],
                             preferred_element_type=jnp.float32) * d

    return pl.pallas_call(
        body,
        grid=(N // _BR,),
        in_specs=[
            pl.BlockSpec((_BR, H), lambda i: (i, 0)),
            pl.BlockSpec((_BR, H), lambda i: (i, 0)),
            pl.BlockSpec((_BR, H), lambda i: (i, 0)),
            pl.BlockSpec((_BR, 1), lambda i: (i, 0)),
            pl.BlockSpec((1, H), lambda i: (0, 0)),
            pl.BlockSpec((H, F2), lambda i: (0, 0)),
        ],
        out_specs=pl.BlockSpec((_BR, F2), lambda i: (i, 0)),
        out_shape=jax.ShapeDtypeStruct((N, F2), jnp.float32),
    )(aa, ab, h0p, dinv, b0, W1p)


def _tc_out(aa, ab, h1p, dinv, b1p):
    def body(aa_ref, ab_ref, h_ref, d_ref, b_ref, o_ref):
        o_ref[...] = (aa_ref[...] + ab_ref[...] + h_ref[...]) * d_ref[...] \
            + b_ref[...]

    return pl.pallas_call(
        body,
        grid=(N // _BR,),
        in_specs=[
            pl.BlockSpec((_BR, F2), lambda i: (i, 0)),
            pl.BlockSpec((_BR, F2), lambda i: (i, 0)),
            pl.BlockSpec((_BR, F2), lambda i: (i, 0)),
            pl.BlockSpec((_BR, 1), lambda i: (i, 0)),
            pl.BlockSpec((1, F2), lambda i: (0, 0)),
        ],
        out_specs=pl.BlockSpec((_BR, F2), lambda i: (i, 0)),
        out_shape=jax.ShapeDtypeStruct((N, F2), jnp.float32),
    )(aa, ab, h1p, dinv, b1p)


def kernel(x, edge_index, W0, b0, W1, b1):
    rowr = edge_index[0].reshape(NW, SEC, KS, CH)
    colr = edge_index[1].reshape(NW, SEC, KS, CH)
    colr_deg = edge_index[1].reshape(NW, K, CH)

    h0 = _tc_mm_raw(x, W0)
    cnt = _sc_degree(colr_deg)
    ca = cnt[0, :N, None]
    cb = cnt[1, :N, None]

    h0p, dinv = _tc_scale(h0, ca, cb)
    agg1 = _sc_agg_h(h0p, rowr, colr)

    W1p = jnp.pad(W1, ((0, 0), (0, F2 - C)))
    b1p = jnp.pad(b1, (0, F2 - C))
    h1p = _tc_mm2(agg1[0, :N], agg1[1, :N], h0p, dinv, b0[None, :], W1p)
    agg2 = _sc_agg_c(h1p, rowr, colr)

    out = _tc_out(agg2[0, :N], agg2[1, :N], h1p, dinv, b1p[None, :])
    return out[:, :C]


# speedup vs baseline: 1.0974x; 1.0974x over previous
import functools
import jax
import jax.numpy as jnp
from jax import lax
from jax.experimental import pallas as pl
from jax.experimental.pallas import tpu as pltpu
from jax.experimental.pallas import tpu_sc as plsc

N = 10000
E = 320000
D = 128
H = 128
C = 40
F2 = 48

NC, NS = 2, 16
NW = NC * NS
CH = 40
K = E // (NW * CH)
SEC = 5
KS = K // SEC
NB = 5
NPAD = 10240
RPT = NPAD // NS

_SC_MESH = dict(core_axis_name="c", subcore_axis_name="s")


@functools.partial(
    pl.kernel,
    out_type=jax.ShapeDtypeStruct((NC, NPAD), jnp.float32),
    mesh=plsc.VectorSubcoreMesh(**_SC_MESH),
    scratch_types=[
        pltpu.VMEM((K, CH), jnp.int32),
        pltpu.VMEM((RPT,), jnp.float32),
        pltpu.VMEM_SHARED((NPAD,), jnp.float32),
        pltpu.SemaphoreType.DMA,
    ],
)
def _sc_degree(colr, out, idx_v, ones_v, acc, sem):
    cid = lax.axis_index("c")
    sid = lax.axis_index("s")
    wid = sid * NC + cid
    r0 = sid * RPT

    def fill(i, carry):
        ones_v[pl.ds(i * 16, 16)] = jnp.zeros((16,), jnp.float32)
        return carry

    lax.fori_loop(0, RPT // 16, fill, 0)
    pltpu.sync_copy(ones_v, acc.at[pl.ds(r0, RPT)])
    for i in range((CH + 15) // 16):
        ones_v[pl.ds(i * 16, 16)] = jnp.full((16,), 1.0, jnp.float32)
    pltpu.sync_copy(colr.at[wid], idx_v)
    plsc.subcore_barrier()

    def fire(j, carry):
        pltpu.async_copy(ones_v.at[pl.ds(0, CH)], acc.at[idx_v.at[j]], sem,
                         add=True)
        return carry

    def drain(j, carry):
        pltpu.make_async_copy(ones_v.at[pl.ds(0, CH)], acc.at[idx_v.at[0]],
                              sem).wait()
        return carry

    lax.fori_loop(0, K, fire, 0)
    lax.fori_loop(0, K, drain, 0)
    plsc.subcore_barrier()
    pltpu.sync_copy(acc.at[pl.ds(r0, RPT)], out.at[cid, pl.ds(r0, RPT)])


def _make_agg(F, tc_tiling=True, dtype=jnp.float32):
    lanes = 32 if dtype == jnp.bfloat16 else 16

    @functools.partial(
        pl.kernel,
        out_type=jax.ShapeDtypeStruct((NC, NPAD, F), dtype),
        mesh=plsc.VectorSubcoreMesh(**_SC_MESH),
        compiler_params=pltpu.CompilerParams(use_tc_tiling_on_sc=tc_tiling),
        scratch_types=[
            pltpu.VMEM((KS, CH), jnp.int32),
            pltpu.VMEM((KS, CH), jnp.int32),
        ] + [pltpu.VMEM((CH, F), dtype) for _ in range(NB)]
          + [pltpu.VMEM_SHARED((NPAD, F), dtype)]
          + [pltpu.SemaphoreType.DMA for _ in range(2 * NB)],
    )
    def agg(h, rowr, colr, out, row_v, col_v, *rest):
        bufs = rest[:NB]
        acc = rest[NB]
        gs = rest[NB + 1:NB + 1 + NB]
        ss = rest[NB + 1 + NB:]
        cid = lax.axis_index("c")
        sid = lax.axis_index("s")
        wid = sid * NC + cid
        r0 = sid * RPT

        def fill(i, carry):
            bufs[0][i // (F // lanes), pl.ds((i % (F // lanes)) * lanes,
                                             lanes)] = \
                jnp.zeros((lanes,), dtype)
            return carry

        lax.fori_loop(0, CH * F // lanes, fill, 0)
        for jj in range(RPT // CH):
            pltpu.sync_copy(bufs[0], acc.at[pl.ds(r0 + jj * CH, CH)])
        plsc.subcore_barrier()

        for s in range(SEC):
            pltpu.sync_copy(rowr.at[wid, s], row_v)
            pltpu.sync_copy(colr.at[wid, s], col_v)
            for i in range(NB):
                pltpu.async_copy(h.at[row_v.at[i]], bufs[i], gs[i])

            def body(g, carry):
                base = NB * g
                for i in range(NB):
                    j = base + i
                    pltpu.make_async_copy(h.at[row_v.at[j]], bufs[i],
                                          gs[i]).wait()
                    pltpu.async_copy(bufs[i], acc.at[col_v.at[j]], ss[i],
                                     add=True)
                for i in range(NB):
                    j = base + NB + i
                    pltpu.make_async_copy(bufs[i], acc.at[col_v.at[j]],
                                          ss[i]).wait()
                    pltpu.async_copy(h.at[row_v.at[j]], bufs[i], gs[i])
                return carry

            lax.fori_loop(0, KS // NB - 1, body, 0)
            for i in range(NB):
                j = KS - NB + i
                pltpu.make_async_copy(h.at[row_v.at[j]], bufs[i],
                                      gs[i]).wait()
                pltpu.async_copy(bufs[i], acc.at[col_v.at[j]], ss[i],
                                 add=True)
            for i in range(NB):
                pltpu.make_async_copy(bufs[i], acc.at[col_v.at[KS - 1]],
                                      ss[i]).wait()
        plsc.subcore_barrier()
        pltpu.sync_copy(acc.at[pl.ds(r0, RPT)],
                        out.at[cid, pl.ds(r0, RPT)])

    return agg


_sc_agg_h = _make_agg(H, tc_tiling=False, dtype=jnp.bfloat16)
_sc_agg_c = _make_agg(F2, tc_tiling=False)


_BR = 1000


def _tc_mm_raw(x, W0):
    def body(x_ref, w_ref, h_ref):
        h_ref[...] = jnp.dot(x_ref[...], w_ref[...],
                             preferred_element_type=jnp.float32)

    return pl.pallas_call(
        body,
        grid=(N // _BR,),
        in_specs=[
            pl.BlockSpec((_BR, D), lambda i: (i, 0)),
            pl.BlockSpec((D, H), lambda i: (0, 0)),
        ],
        out_specs=pl.BlockSpec((_BR, H), lambda i: (i, 0)),
        out_shape=jax.ShapeDtypeStruct((N, H), jnp.float32),
    )(x, W0)


def _tc_scale(h0, ca, cb):
    def body(h0_ref, ca_ref, cb_ref, h_ref, hb_ref, d_ref):
        d = lax.rsqrt(ca_ref[...] + cb_ref[...] + 1.0)
        hp = h0_ref[...] * d
        h_ref[...] = hp
        hb_ref[...] = hp.astype(jnp.bfloat16)
        d_ref[...] = d

    return pl.pallas_call(
        body,
        grid=(N // _BR,),
        in_specs=[
            pl.BlockSpec((_BR, H), lambda i: (i, 0)),
            pl.BlockSpec((_BR, 1), lambda i: (i, 0)),
            pl.BlockSpec((_BR, 1), lambda i: (i, 0)),
        ],
        out_specs=[
            pl.BlockSpec((_BR, H), lambda i: (i, 0)),
            pl.BlockSpec((_BR, H), lambda i: (i, 0)),
            pl.BlockSpec((_BR, 1), lambda i: (i, 0)),
        ],
        out_shape=[
            jax.ShapeDtypeStruct((N, H), jnp.float32),
            jax.ShapeDtypeStruct((N, H), jnp.bfloat16),
            jax.ShapeDtypeStruct((N, 1), jnp.float32),
        ],
    )(h0, ca, cb)


def _tc_mm2(aa, ab, h0p, dinv, b0, W1p):
    def body(aa_ref, ab_ref, h_ref, d_ref, b_ref, w_ref, o_ref):
        d = d_ref[...]
        neigh = aa_ref[...].astype(jnp.float32) \
            + ab_ref[...].astype(jnp.float32)
        z = jnp.maximum((neigh + h_ref[...]) * d + b_ref[...], 0.0)
        o_ref[...] = jnp.dot(z, w_ref[...],
                             preferred_element_type=jnp.float32) * d

    return pl.pallas_call(
        body,
        grid=(N // _BR,),
        in_specs=[
            pl.BlockSpec((_BR, H), lambda i: (i, 0)),
            pl.BlockSpec((_BR, H), lambda i: (i, 0)),
            pl.BlockSpec((_BR, H), lambda i: (i, 0)),
            pl.BlockSpec((_BR, 1), lambda i: (i, 0)),
            pl.BlockSpec((1, H), lambda i: (0, 0)),
            pl.BlockSpec((H, F2), lambda i: (0, 0)),
        ],
        out_specs=pl.BlockSpec((_BR, F2), lambda i: (i, 0)),
        out_shape=jax.ShapeDtypeStruct((N, F2), jnp.float32),
    )(aa, ab, h0p, dinv, b0, W1p)


def _tc_out(aa, ab, h1p, dinv, b1p):
    def body(aa_ref, ab_ref, h_ref, d_ref, b_ref, o_ref):
        o_ref[...] = (aa_ref[...] + ab_ref[...] + h_ref[...]) * d_ref[...] \
            + b_ref[...]

    return pl.pallas_call(
        body,
        grid=(N // _BR,),
        in_specs=[
            pl.BlockSpec((_BR, F2), lambda i: (i, 0)),
            pl.BlockSpec((_BR, F2), lambda i: (i, 0)),
            pl.BlockSpec((_BR, F2), lambda i: (i, 0)),
            pl.BlockSpec((_BR, 1), lambda i: (i, 0)),
            pl.BlockSpec((1, F2), lambda i: (0, 0)),
        ],
        out_specs=pl.BlockSpec((_BR, F2), lambda i: (i, 0)),
        out_shape=jax.ShapeDtypeStruct((N, F2), jnp.float32),
    )(aa, ab, h1p, dinv, b1p)


def kernel(x, edge_index, W0, b0, W1, b1):
    rowr = edge_index[0].reshape(NW, SEC, KS, CH)
    colr = edge_index[1].reshape(NW, SEC, KS, CH)
    colr_deg = edge_index[1].reshape(NW, K, CH)

    h0 = _tc_mm_raw(x, W0)
    cnt = _sc_degree(colr_deg)
    ca = cnt[0, :N, None]
    cb = cnt[1, :N, None]

    h0p, h0pb, dinv = _tc_scale(h0, ca, cb)
    agg1 = _sc_agg_h(h0pb, rowr, colr)

    W1p = jnp.pad(W1, ((0, 0), (0, F2 - C)))
    b1p = jnp.pad(b1, (0, F2 - C))
    h1p = _tc_mm2(agg1[0, :N], agg1[1, :N], h0p, dinv, b0[None, :], W1p)
    agg2 = _sc_agg_c(h1p, rowr, colr)

    out = _tc_out(agg2[0, :N], agg2[1, :N], h1p, dinv, b1p[None, :])
    return out[:, :C]


# speedup vs baseline: 1.1305x; 1.0302x over previous
import functools
import jax
import jax.numpy as jnp
from jax import lax
from jax.experimental import pallas as pl
from jax.experimental.pallas import tpu as pltpu
from jax.experimental.pallas import tpu_sc as plsc

N = 10000
E = 320000
D = 128
H = 128
C = 40
F2 = 64

NC, NS = 2, 16
NW = NC * NS
CH = 40
K = E // (NW * CH)
SEC = 5
KS = K // SEC
NB = 5
NPAD = 10240
RPT = NPAD // NS

_SC_MESH = dict(core_axis_name="c", subcore_axis_name="s")


@functools.partial(
    pl.kernel,
    out_type=jax.ShapeDtypeStruct((NC, NPAD), jnp.float32),
    mesh=plsc.VectorSubcoreMesh(**_SC_MESH),
    scratch_types=[
        pltpu.VMEM((K, CH), jnp.int32),
        pltpu.VMEM((RPT,), jnp.float32),
        pltpu.VMEM_SHARED((NPAD,), jnp.float32),
        pltpu.SemaphoreType.DMA,
    ],
)
def _sc_degree(colr, out, idx_v, ones_v, acc, sem):
    cid = lax.axis_index("c")
    sid = lax.axis_index("s")
    wid = sid * NC + cid
    r0 = sid * RPT

    def fill(i, carry):
        ones_v[pl.ds(i * 16, 16)] = jnp.zeros((16,), jnp.float32)
        return carry

    lax.fori_loop(0, RPT // 16, fill, 0)
    pltpu.sync_copy(ones_v, acc.at[pl.ds(r0, RPT)])
    for i in range((CH + 15) // 16):
        ones_v[pl.ds(i * 16, 16)] = jnp.full((16,), 1.0, jnp.float32)
    pltpu.sync_copy(colr.at[wid], idx_v)
    plsc.subcore_barrier()

    def fire(j, carry):
        pltpu.async_copy(ones_v.at[pl.ds(0, CH)], acc.at[idx_v.at[j]], sem,
                         add=True)
        return carry

    def drain(j, carry):
        pltpu.make_async_copy(ones_v.at[pl.ds(0, CH)], acc.at[idx_v.at[0]],
                              sem).wait()
        return carry

    lax.fori_loop(0, K, fire, 0)
    lax.fori_loop(0, K, drain, 0)
    plsc.subcore_barrier()
    pltpu.sync_copy(acc.at[pl.ds(r0, RPT)], out.at[cid, pl.ds(r0, RPT)])


def _make_agg(F, tc_tiling=True, dtype=jnp.float32):
    lanes = 32 if dtype == jnp.bfloat16 else 16

    @functools.partial(
        pl.kernel,
        out_type=jax.ShapeDtypeStruct((NC, NPAD, F), dtype),
        mesh=plsc.VectorSubcoreMesh(**_SC_MESH),
        compiler_params=pltpu.CompilerParams(use_tc_tiling_on_sc=tc_tiling),
        scratch_types=[
            pltpu.VMEM((KS, CH), jnp.int32),
            pltpu.VMEM((KS, CH), jnp.int32),
        ] + [pltpu.VMEM((CH, F), dtype) for _ in range(NB)]
          + [pltpu.VMEM_SHARED((NPAD, F), dtype)]
          + [pltpu.SemaphoreType.DMA for _ in range(2 * NB)],
    )
    def agg(h, rowr, colr, out, row_v, col_v, *rest):
        bufs = rest[:NB]
        acc = rest[NB]
        gs = rest[NB + 1:NB + 1 + NB]
        ss = rest[NB + 1 + NB:]
        cid = lax.axis_index("c")
        sid = lax.axis_index("s")
        wid = sid * NC + cid
        r0 = sid * RPT

        def fill(i, carry):
            bufs[0][i // (F // lanes), pl.ds((i % (F // lanes)) * lanes,
                                             lanes)] = \
                jnp.zeros((lanes,), dtype)
            return carry

        lax.fori_loop(0, CH * F // lanes, fill, 0)
        for jj in range(RPT // CH):
            pltpu.sync_copy(bufs[0], acc.at[pl.ds(r0 + jj * CH, CH)])
        plsc.subcore_barrier()

        for s in range(SEC):
            pltpu.sync_copy(rowr.at[wid, s], row_v)
            pltpu.sync_copy(colr.at[wid, s], col_v)
            for i in range(NB):
                pltpu.async_copy(h.at[row_v.at[i]], bufs[i], gs[i])

            def body(g, carry):
                base = NB * g
                for i in range(NB):
                    j = base + i
                    pltpu.make_async_copy(h.at[row_v.at[j]], bufs[i],
                                          gs[i]).wait()
                    pltpu.async_copy(bufs[i], acc.at[col_v.at[j]], ss[i],
                                     add=True)
                for i in range(NB):
                    j = base + NB + i
                    pltpu.make_async_copy(bufs[i], acc.at[col_v.at[j]],
                                          ss[i]).wait()
                    pltpu.async_copy(h.at[row_v.at[j]], bufs[i], gs[i])
                return carry

            lax.fori_loop(0, KS // NB - 1, body, 0)
            for i in range(NB):
                j = KS - NB + i
                pltpu.make_async_copy(h.at[row_v.at[j]], bufs[i],
                                      gs[i]).wait()
                pltpu.async_copy(bufs[i], acc.at[col_v.at[j]], ss[i],
                                 add=True)
            for i in range(NB):
                pltpu.make_async_copy(bufs[i], acc.at[col_v.at[KS - 1]],
                                      ss[i]).wait()
        plsc.subcore_barrier()
        pltpu.sync_copy(acc.at[pl.ds(r0, RPT)],
                        out.at[cid, pl.ds(r0, RPT)])

    return agg


_sc_agg_h = _make_agg(H, tc_tiling=False, dtype=jnp.bfloat16)
_sc_agg_c = _make_agg(F2, tc_tiling=False, dtype=jnp.bfloat16)


_BR = 1000


def _tc_mm_raw(x, W0):
    def body(x_ref, w_ref, h_ref):
        h_ref[...] = jnp.dot(x_ref[...], w_ref[...],
                             preferred_element_type=jnp.float32)

    return pl.pallas_call(
        body,
        grid=(N // _BR,),
        in_specs=[
            pl.BlockSpec((_BR, D), lambda i: (i, 0)),
            pl.BlockSpec((D, H), lambda i: (0, 0)),
        ],
        out_specs=pl.BlockSpec((_BR, H), lambda i: (i, 0)),
        out_shape=jax.ShapeDtypeStruct((N, H), jnp.float32),
    )(x, W0)


def _tc_scale(h0, ca, cb):
    def body(h0_ref, ca_ref, cb_ref, h_ref, hb_ref, d_ref):
        d = lax.rsqrt(ca_ref[...] + cb_ref[...] + 1.0)
        hp = h0_ref[...] * d
        h_ref[...] = hp
        hb_ref[...] = hp.astype(jnp.bfloat16)
        d_ref[...] = d

    return pl.pallas_call(
        body,
        grid=(N // _BR,),
        in_specs=[
            pl.BlockSpec((_BR, H), lambda i: (i, 0)),
            pl.BlockSpec((_BR, 1), lambda i: (i, 0)),
            pl.BlockSpec((_BR, 1), lambda i: (i, 0)),
        ],
        out_specs=[
            pl.BlockSpec((_BR, H), lambda i: (i, 0)),
            pl.BlockSpec((_BR, H), lambda i: (i, 0)),
            pl.BlockSpec((_BR, 1), lambda i: (i, 0)),
        ],
        out_shape=[
            jax.ShapeDtypeStruct((N, H), jnp.float32),
            jax.ShapeDtypeStruct((N, H), jnp.bfloat16),
            jax.ShapeDtypeStruct((N, 1), jnp.float32),
        ],
    )(h0, ca, cb)


def _tc_mm2(aa, ab, h0p, dinv, b0, W1p):
    def body(aa_ref, ab_ref, h_ref, d_ref, b_ref, w_ref, o_ref, ob_ref):
        d = d_ref[...]
        neigh = aa_ref[...].astype(jnp.float32) \
            + ab_ref[...].astype(jnp.float32)
        z = jnp.maximum((neigh + h_ref[...]) * d + b_ref[...], 0.0)
        o = jnp.dot(z, w_ref[...],
                    preferred_element_type=jnp.float32) * d
        o_ref[...] = o
        ob_ref[...] = o.astype(jnp.bfloat16)

    return pl.pallas_call(
        body,
        grid=(N // _BR,),
        in_specs=[
            pl.BlockSpec((_BR, H), lambda i: (i, 0)),
            pl.BlockSpec((_BR, H), lambda i: (i, 0)),
            pl.BlockSpec((_BR, H), lambda i: (i, 0)),
            pl.BlockSpec((_BR, 1), lambda i: (i, 0)),
            pl.BlockSpec((1, H), lambda i: (0, 0)),
            pl.BlockSpec((H, F2), lambda i: (0, 0)),
        ],
        out_specs=[
            pl.BlockSpec((_BR, F2), lambda i: (i, 0)),
            pl.BlockSpec((_BR, F2), lambda i: (i, 0)),
        ],
        out_shape=[
            jax.ShapeDtypeStruct((N, F2), jnp.float32),
            jax.ShapeDtypeStruct((N, F2), jnp.bfloat16),
        ],
    )(aa, ab, h0p, dinv, b0, W1p)


def _tc_out(aa, ab, h1p, dinv, b1p):
    def body(aa_ref, ab_ref, h_ref, d_ref, b_ref, o_ref):
        neigh = aa_ref[...].astype(jnp.float32) \
            + ab_ref[...].astype(jnp.float32)
        o_ref[...] = (neigh + h_ref[...]) * d_ref[...] + b_ref[...]

    return pl.pallas_call(
        body,
        grid=(N // _BR,),
        in_specs=[
            pl.BlockSpec((_BR, F2), lambda i: (i, 0)),
            pl.BlockSpec((_BR, F2), lambda i: (i, 0)),
            pl.BlockSpec((_BR, F2), lambda i: (i, 0)),
            pl.BlockSpec((_BR, 1), lambda i: (i, 0)),
            pl.BlockSpec((1, F2), lambda i: (0, 0)),
        ],
        out_specs=pl.BlockSpec((_BR, F2), lambda i: (i, 0)),
        out_shape=jax.ShapeDtypeStruct((N, F2), jnp.float32),
    )(aa, ab, h1p, dinv, b1p)


def kernel(x, edge_index, W0, b0, W1, b1):
    rowr = edge_index[0].reshape(NW, SEC, KS, CH)
    colr = edge_index[1].reshape(NW, SEC, KS, CH)
    colr_deg = edge_index[1].reshape(NW, K, CH)

    h0 = _tc_mm_raw(x, W0)
    cnt = _sc_degree(colr_deg)
    ca = cnt[0, :N, None]
    cb = cnt[1, :N, None]

    h0p, h0pb, dinv = _tc_scale(h0, ca, cb)
    agg1 = _sc_agg_h(h0pb, rowr, colr)

    W1p = jnp.pad(W1, ((0, 0), (0, F2 - C)))
    b1p = jnp.pad(b1, (0, F2 - C))
    h1p, h1pb = _tc_mm2(agg1[0, :N], agg1[1, :N], h0p, dinv,
                        b0[None, :], W1p)
    agg2 = _sc_agg_c(h1pb, rowr, colr)

    out = _tc_out(agg2[0, :N], agg2[1, :N], h1p, dinv, b1p[None, :])
    return out[:, :C]


# speedup vs baseline: 1.1344x; 1.0035x over previous
import functools
import jax
import jax.numpy as jnp
from jax import lax
from jax.experimental import pallas as pl
from jax.experimental.pallas import tpu as pltpu
from jax.experimental.pallas import tpu_sc as plsc

N = 10000
E = 320000
D = 128
H = 128
C = 40
F2 = 64

NC, NS = 2, 16
NW = NC * NS
CH = 40
K = E // (NW * CH)
SEC = 5
KS = K // SEC
NB = 5
NPAD = 10240
RPT = NPAD // NS

_SC_MESH = dict(core_axis_name="c", subcore_axis_name="s")


@functools.partial(
    pl.kernel,
    out_type=jax.ShapeDtypeStruct((NC, NPAD), jnp.float32),
    mesh=plsc.VectorSubcoreMesh(**_SC_MESH),
    scratch_types=[
        pltpu.VMEM((K, CH), jnp.int32),
        pltpu.VMEM((RPT,), jnp.float32),
        pltpu.VMEM_SHARED((NPAD,), jnp.float32),
        pltpu.SemaphoreType.DMA,
    ],
)
def _sc_degree(colr, out, idx_v, ones_v, acc, sem):
    cid = lax.axis_index("c")
    sid = lax.axis_index("s")
    wid = sid * NC + cid
    r0 = sid * RPT

    def fill(i, carry):
        ones_v[pl.ds(i * 16, 16)] = jnp.zeros((16,), jnp.float32)
        return carry

    lax.fori_loop(0, RPT // 16, fill, 0)
    pltpu.sync_copy(ones_v, acc.at[pl.ds(r0, RPT)])
    for i in range((CH + 15) // 16):
        ones_v[pl.ds(i * 16, 16)] = jnp.full((16,), 1.0, jnp.float32)
    pltpu.sync_copy(colr.at[wid], idx_v)
    plsc.subcore_barrier()

    def fire(j, carry):
        pltpu.async_copy(ones_v.at[pl.ds(0, CH)], acc.at[idx_v.at[j]], sem,
                         add=True)
        return carry

    def drain(j, carry):
        pltpu.make_async_copy(ones_v.at[pl.ds(0, CH)], acc.at[idx_v.at[0]],
                              sem).wait()
        return carry

    lax.fori_loop(0, K, fire, 0)
    lax.fori_loop(0, K, drain, 0)
    plsc.subcore_barrier()
    pltpu.sync_copy(acc.at[pl.ds(r0, RPT)], out.at[cid, pl.ds(r0, RPT)])


def _make_agg(F, tc_tiling=True, dtype=jnp.float32):
    lanes = 32 if dtype == jnp.bfloat16 else 16

    @functools.partial(
        pl.kernel,
        out_type=jax.ShapeDtypeStruct((NC, NPAD, F), dtype),
        mesh=plsc.VectorSubcoreMesh(**_SC_MESH),
        compiler_params=pltpu.CompilerParams(use_tc_tiling_on_sc=tc_tiling),
        scratch_types=[
            pltpu.VMEM((KS, CH), jnp.int32),
            pltpu.VMEM((KS, CH), jnp.int32),
        ] + [pltpu.VMEM((CH, F), dtype) for _ in range(NB)]
          + [pltpu.VMEM_SHARED((NPAD, F), dtype)]
          + [pltpu.SemaphoreType.DMA for _ in range(2 * NB)],
    )
    def agg(h, rowr, colr, out, row_v, col_v, *rest):
        bufs = rest[:NB]
        acc = rest[NB]
        gs = rest[NB + 1:NB + 1 + NB]
        ss = rest[NB + 1 + NB:]
        cid = lax.axis_index("c")
        sid = lax.axis_index("s")
        wid = sid * NC + cid
        r0 = sid * RPT

        def fill(i, carry):
            bufs[0][i // (F // lanes), pl.ds((i % (F // lanes)) * lanes,
                                             lanes)] = \
                jnp.zeros((lanes,), dtype)
            return carry

        lax.fori_loop(0, CH * F // lanes, fill, 0)
        for jj in range(RPT // CH):
            pltpu.sync_copy(bufs[0], acc.at[pl.ds(r0 + jj * CH, CH)])
        plsc.subcore_barrier()

        for s in range(SEC):
            pltpu.sync_copy(rowr.at[wid, s], row_v)
            pltpu.sync_copy(colr.at[wid, s], col_v)
            for i in range(NB):
                pltpu.async_copy(h.at[row_v.at[i]], bufs[i], gs[i])

            def body(g, carry):
                base = NB * g
                for i in range(NB):
                    j = base + i
                    pltpu.make_async_copy(h.at[row_v.at[j]], bufs[i],
                                          gs[i]).wait()
                    pltpu.async_copy(bufs[i], acc.at[col_v.at[j]], ss[i],
                                     add=True)
                for i in range(NB):
                    j = base + NB + i
                    pltpu.make_async_copy(bufs[i], acc.at[col_v.at[j]],
                                          ss[i]).wait()
                    pltpu.async_copy(h.at[row_v.at[j]], bufs[i], gs[i])
                return carry

            lax.fori_loop(0, KS // NB - 1, body, 0)
            for i in range(NB):
                j = KS - NB + i
                pltpu.make_async_copy(h.at[row_v.at[j]], bufs[i],
                                      gs[i]).wait()
                pltpu.async_copy(bufs[i], acc.at[col_v.at[j]], ss[i],
                                 add=True)
            for i in range(NB):
                pltpu.make_async_copy(bufs[i], acc.at[col_v.at[KS - 1]],
                                      ss[i]).wait()
        plsc.subcore_barrier()
        pltpu.sync_copy(acc.at[pl.ds(r0, RPT)],
                        out.at[cid, pl.ds(r0, RPT)])

    return agg


_sc_agg_h = _make_agg(H, tc_tiling=False, dtype=jnp.bfloat16)
_sc_agg_c = _make_agg(F2, tc_tiling=False, dtype=jnp.bfloat16)


_BR = 1000


def _tc_mm1(x, W0, ca, cb):
    def body(x_ref, w_ref, ca_ref, cb_ref, h_ref, hb_ref, d_ref):
        d = lax.rsqrt(ca_ref[...] + cb_ref[...] + 1.0)
        hp = jnp.dot(x_ref[...], w_ref[...],
                     preferred_element_type=jnp.float32) * d
        h_ref[...] = hp
        hb_ref[...] = hp.astype(jnp.bfloat16)
        d_ref[...] = d

    return pl.pallas_call(
        body,
        grid=(N // _BR,),
        in_specs=[
            pl.BlockSpec((_BR, D), lambda i: (i, 0)),
            pl.BlockSpec((D, H), lambda i: (0, 0)),
            pl.BlockSpec((_BR, 1), lambda i: (i, 0)),
            pl.BlockSpec((_BR, 1), lambda i: (i, 0)),
        ],
        out_specs=[
            pl.BlockSpec((_BR, H), lambda i: (i, 0)),
            pl.BlockSpec((_BR, H), lambda i: (i, 0)),
            pl.BlockSpec((_BR, 1), lambda i: (i, 0)),
        ],
        out_shape=[
            jax.ShapeDtypeStruct((N, H), jnp.float32),
            jax.ShapeDtypeStruct((N, H), jnp.bfloat16),
            jax.ShapeDtypeStruct((N, 1), jnp.float32),
        ],
    )(x, W0, ca, cb)


def _tc_mm2(aa, ab, h0p, dinv, b0, W1p):
    def body(aa_ref, ab_ref, h_ref, d_ref, b_ref, w_ref, o_ref, ob_ref):
        d = d_ref[...]
        neigh = aa_ref[...].astype(jnp.float32) \
            + ab_ref[...].astype(jnp.float32)
        z = jnp.maximum((neigh + h_ref[...]) * d + b_ref[...], 0.0)
        o = jnp.dot(z, w_ref[...],
                    preferred_element_type=jnp.float32) * d
        o_ref[...] = o
        ob_ref[...] = o.astype(jnp.bfloat16)

    return pl.pallas_call(
        body,
        grid=(N // _BR,),
        in_specs=[
            pl.BlockSpec((_BR, H), lambda i: (i, 0)),
            pl.BlockSpec((_BR, H), lambda i: (i, 0)),
            pl.BlockSpec((_BR, H), lambda i: (i, 0)),
            pl.BlockSpec((_BR, 1), lambda i: (i, 0)),
            pl.BlockSpec((1, H), lambda i: (0, 0)),
            pl.BlockSpec((H, F2), lambda i: (0, 0)),
        ],
        out_specs=[
            pl.BlockSpec((_BR, F2), lambda i: (i, 0)),
            pl.BlockSpec((_BR, F2), lambda i: (i, 0)),
        ],
        out_shape=[
            jax.ShapeDtypeStruct((N, F2), jnp.float32),
            jax.ShapeDtypeStruct((N, F2), jnp.bfloat16),
        ],
    )(aa, ab, h0p, dinv, b0, W1p)


def _tc_out(aa, ab, h1p, dinv, b1p):
    def body(aa_ref, ab_ref, h_ref, d_ref, b_ref, o_ref):
        neigh = aa_ref[...].astype(jnp.float32) \
            + ab_ref[...].astype(jnp.float32)
        o_ref[...] = (neigh + h_ref[...]) * d_ref[...] + b_ref[...]

    return pl.pallas_call(
        body,
        grid=(N // _BR,),
        in_specs=[
            pl.BlockSpec((_BR, F2), lambda i: (i, 0)),
            pl.BlockSpec((_BR, F2), lambda i: (i, 0)),
            pl.BlockSpec((_BR, F2), lambda i: (i, 0)),
            pl.BlockSpec((_BR, 1), lambda i: (i, 0)),
            pl.BlockSpec((1, F2), lambda i: (0, 0)),
        ],
        out_specs=pl.BlockSpec((_BR, F2), lambda i: (i, 0)),
        out_shape=jax.ShapeDtypeStruct((N, F2), jnp.float32),
    )(aa, ab, h1p, dinv, b1p)


def kernel(x, edge_index, W0, b0, W1, b1):
    rowr = edge_index[0].reshape(NW, SEC, KS, CH)
    colr = edge_index[1].reshape(NW, SEC, KS, CH)
    colr_deg = edge_index[1].reshape(NW, K, CH)

    cnt = _sc_degree(colr_deg)
    ca = cnt[0, :N, None]
    cb = cnt[1, :N, None]

    h0p, h0pb, dinv = _tc_mm1(x, W0, ca, cb)
    agg1 = _sc_agg_h(h0pb, rowr, colr)

    W1p = jnp.pad(W1, ((0, 0), (0, F2 - C)))
    b1p = jnp.pad(b1, (0, F2 - C))
    h1p, h1pb = _tc_mm2(agg1[0, :N], agg1[1, :N], h0p, dinv,
                        b0[None, :], W1p)
    agg2 = _sc_agg_c(h1pb, rowr, colr)

    out = _tc_out(agg2[0, :N], agg2[1, :N], h1p, dinv, b1p[None, :])
    return out[:, :C]
